# baseline (speedup 1.0000x reference)
import functools

import jax
import jax.numpy as jnp
from jax import lax
from jax.experimental import pallas as pl
from jax.experimental.pallas import tpu as pltpu
from jax.experimental.pallas import tpu_sc as plsc

_B = 4096
_C = 1000
_CPAD = 1024
_BLK = 128
_NC = 2
_NS = 16
_NW = _NC * _NS
_LANES = 16
_SLABS_PER_W = _CPAD // (_NW * _LANES)
_CHUNK = _B // 4
_UNROLL = 16


def _tc_body(pred_ref, targ_ref, bits_ref, mino_ref):
    cls = (pl.program_id(0) * _BLK
           + lax.broadcasted_iota(jnp.int32, (1, _BLK), 1))
    valid = cls < _C
    p = jnp.where(valid, pred_ref[...], 0.0)
    t = jnp.where(valid, targ_ref[...], 0.0)
    loss = jnp.maximum(p, 0.0) - p * t + jnp.log1p(jnp.exp(-jnp.abs(p)))

    pos = jnp.sum(t, axis=0, keepdims=True)
    pos_gt = (pos * 2.0 >= float(_B)).astype(jnp.float32)
    majority = t == pos_gt

    mino = jnp.sum(jnp.where(majority, 0.0, loss))
    bits = jnp.where(majority, lax.bitcast_convert_type(loss, jnp.int32),
                     jnp.int32(-1))
    bits_ref[...] = jnp.transpose(bits, (1, 0))

    @pl.when(pl.program_id(0) == 0)
    def _():
        mino_ref[...] = jnp.zeros((1, 1), jnp.float32)

    mino_ref[...] += jnp.reshape(mino, (1, 1))


def _tc_stage(predp, targp):
    return pl.pallas_call(
        _tc_body,
        grid=(_CPAD // _BLK,),
        in_specs=[
            pl.BlockSpec((_B, _BLK), lambda j: (0, j)),
            pl.BlockSpec((_B, _BLK), lambda j: (0, j)),
        ],
        out_specs=[
            pl.BlockSpec((_BLK, _B), lambda j: (j, 0)),
            pl.BlockSpec((1, 1), lambda j: (0, 0)),
        ],
        out_shape=[
            jax.ShapeDtypeStruct((_CPAD, _B), jnp.int32),
            jax.ShapeDtypeStruct((1, 1), jnp.float32),
        ],
        compiler_params=pltpu.CompilerParams(
            dimension_semantics=("arbitrary",),
        ),
    )(predp, targp)


def _sc_body(bits_hbm, out_hbm, stage_ref, slab_ref, hist_ref, res_ref):
    wid = lax.axis_index("s") * _NC + lax.axis_index("c")
    lanes = jnp.arange(_LANES, dtype=jnp.int32)
    zero_i = jnp.zeros((_LANES,), jnp.int32)
    one_i = jnp.ones((_LANES,), jnp.int32)
    zero_f = jnp.zeros((_LANES,), jnp.float32)

    for slab in range(_SLABS_PER_W):
        c0 = (wid * _SLABS_PER_W + slab) * _LANES

        for h in range(4):
            pltpu.sync_copy(
                bits_hbm.at[pl.ds(c0, _LANES), pl.ds(h * _CHUNK, _CHUNK)],
                stage_ref)
            for c in range(_LANES):
                col = jnp.full((_LANES,), c, jnp.int32)

                @plsc.parallel_loop(0, _CHUNK // _LANES, unroll=8)
                def _(i, c=c, col=col, h=h):
                    v = stage_ref[c, pl.ds(i * _LANES, _LANES)]
                    rows = (h * _CHUNK + i * _LANES) + lanes
                    plsc.store_scatter(slab_ref, [rows * _LANES + col], v)

        P = zero_i
        A = zero_i
        k_rem = zero_i
        k_vec = zero_i
        for li, (ps, shift, nbins) in enumerate(
                ((31, 20, 2048), (20, 10, 1024), (10, 0, 1024))):

            @plsc.parallel_loop(0, nbins, unroll=8)
            def _(i):
                hist_ref[pl.ds(i * _LANES, _LANES)] = zero_i

            if li == 0:
                @plsc.parallel_loop(0, _B // _UNROLL, carry=zero_i)
                def k_vec(i, kk):
                    for j in range(_UNROLL):
                        v = slab_ref[pl.ds((i * _UNROLL + j) * _LANES,
                                           _LANES)]
                        m = v >= 0
                        bn = (v >> 20) & (nbins - 1)
                        plsc.addupdate_scatter(
                            hist_ref, [bn * _LANES + lanes], one_i, mask=m)
                        kk = kk + jnp.where(v < 0, one_i, zero_i)
                    return kk
                k_rem = k_vec
            else:
                @plsc.parallel_loop(0, _B // _UNROLL)
                def _(i, ps=ps, shift=shift, nbins=nbins, P=P):
                    for j in range(_UNROLL):
                        v = slab_ref[pl.ds((i * _UNROLL + j) * _LANES,
                                           _LANES)]
                        m = (v >> ps) == P
                        bn = (v >> shift) & (nbins - 1)
                        plsc.addupdate_scatter(
                            hist_ref, [bn * _LANES + lanes], one_i, mask=m)

            def scan_chunk(i, c, nbins=nbins, k_rem=k_rem):
                S, Al, bst, fnd = c
                for j in range(8):
                    bi_expr = nbins - 1 - (i * 8 + j)
                    h = hist_ref[pl.ds(bi_expr * _LANES, _LANES)]
                    Sn = S + h
                    crossing = (Sn >= k_rem) & (fnd == 0)
                    bst = jnp.where(crossing, bi_expr, bst)
                    Al = jnp.where(crossing, S, Al)
                    fnd = jnp.where(crossing, one_i, fnd)
                    S = Sn
                return (S, Al, bst, fnd)

            _, Al, bst, _ = lax.fori_loop(
                0, nbins // 8, scan_chunk,
                (zero_i, zero_i, zero_i, zero_i), unroll=False)

            k_rem = k_rem - Al
            A = A + Al
            P = (P << (ps - shift)) | bst

        T = P

        @plsc.parallel_loop(0, _B // _UNROLL, carry=zero_f)
        def s(i, acc):
            for j in range(_UNROLL):
                v = slab_ref[pl.ds((i * _UNROLL + j) * _LANES, _LANES)]
                gt = v > T
                acc = acc + jnp.where(gt, plsc.bitcast(v, jnp.float32),
                                      zero_f)
            return acc
        tie = plsc.bitcast(T, jnp.float32)
        extra = (k_vec - A).astype(jnp.float32) * tie
        res_ref[...] = jnp.where(k_vec > 0, s + extra, zero_f)
        pltpu.sync_copy(res_ref, out_hbm.at[pl.ds(c0, _LANES)])


_sc_stage = functools.partial(
    pl.kernel,
    out_type=jax.ShapeDtypeStruct((_CPAD,), jnp.float32),
    mesh=plsc.VectorSubcoreMesh(core_axis_name="c", subcore_axis_name="s"),
    compiler_params=pltpu.CompilerParams(needs_layout_passes=False),
    scratch_types=[
        pltpu.VMEM((_LANES, _CHUNK), jnp.int32),
        pltpu.VMEM((_B * _LANES,), jnp.int32),
        pltpu.VMEM((2048 * _LANES,), jnp.int32),
        pltpu.VMEM((_LANES,), jnp.float32),
    ],
)(_sc_body)


def kernel(pred, target):
    bits, mino = _tc_stage(pred, target)
    topk = _sc_stage(bits)
    return (mino[0, 0] + jnp.sum(topk)) / jnp.float32(_B * _C)


# speedup vs baseline: 1.0534x; 1.0534x over previous
import functools

import jax
import jax.numpy as jnp
from jax import lax
from jax.experimental import pallas as pl
from jax.experimental.pallas import tpu as pltpu
from jax.experimental.pallas import tpu_sc as plsc

_B = 4096
_C = 1000
_CPAD = 1024
_BLK = 128
_NC = 2
_NS = 16
_NW = _NC * _NS
_LANES = 16
_CHALF = _CPAD // 2
_SLABS_PER_W = _CHALF // (_NW * _LANES)
_HALF = _B // 2
_UNROLL = 16


def _tc_body(pred_ref, targ_ref, bits_ref, mino_ref, *, j0):
    cls = ((pl.program_id(0) + j0) * _BLK
           + lax.broadcasted_iota(jnp.int32, (1, _BLK), 1))
    valid = cls < _C
    p = jnp.where(valid, pred_ref[...], 0.0)
    t = jnp.where(valid, targ_ref[...], 0.0)
    loss = jnp.maximum(p, 0.0) - p * t + jnp.log1p(jnp.exp(-jnp.abs(p)))

    pos = jnp.sum(t, axis=0, keepdims=True)
    pos_gt = (pos * 2.0 >= float(_B)).astype(jnp.float32)
    majority = t == pos_gt

    mino = jnp.sum(jnp.where(majority, 0.0, loss))
    bits = jnp.where(majority, lax.bitcast_convert_type(loss, jnp.int32),
                     jnp.int32(-1))
    bits_ref[...] = jnp.transpose(bits, (1, 0))

    @pl.when(pl.program_id(0) == 0)
    def _():
        mino_ref[...] = jnp.zeros((1, 1), jnp.float32)

    mino_ref[...] += jnp.reshape(mino, (1, 1))


def _tc_stage(predp, targp, j0):
    return pl.pallas_call(
        functools.partial(_tc_body, j0=j0),
        grid=(_CHALF // _BLK,),
        in_specs=[
            pl.BlockSpec((_B, _BLK), lambda j, j0=j0: (0, j + j0)),
            pl.BlockSpec((_B, _BLK), lambda j, j0=j0: (0, j + j0)),
        ],
        out_specs=[
            pl.BlockSpec((_BLK, _B), lambda j: (j, 0)),
            pl.BlockSpec((1, 1), lambda j: (0, 0)),
        ],
        out_shape=[
            jax.ShapeDtypeStruct((_CHALF, _B), jnp.int32),
            jax.ShapeDtypeStruct((1, 1), jnp.float32),
        ],
        compiler_params=pltpu.CompilerParams(
            dimension_semantics=("arbitrary",),
        ),
    )(predp, targp)


def _sc_body(bits_hbm, out_hbm, stage_ref, slab_ref, hist_ref, res_ref):
    wid = lax.axis_index("s") * _NC + lax.axis_index("c")
    lanes = jnp.arange(_LANES, dtype=jnp.int32)
    zero_i = jnp.zeros((_LANES,), jnp.int32)
    one_i = jnp.ones((_LANES,), jnp.int32)
    zero_f = jnp.zeros((_LANES,), jnp.float32)

    for slab in range(_SLABS_PER_W):
        c0 = (wid * _SLABS_PER_W + slab) * _LANES

        for h in range(2):
            pltpu.sync_copy(
                bits_hbm.at[pl.ds(c0, _LANES), pl.ds(h * _HALF, _HALF)],
                stage_ref)
            for c in range(_LANES):
                col = jnp.full((_LANES,), c, jnp.int32)

                @plsc.parallel_loop(0, _HALF // _LANES, unroll=8)
                def _(i, c=c, col=col, h=h):
                    v = stage_ref[c, pl.ds(i * _LANES, _LANES)]
                    rows = (h * _HALF + i * _LANES) + lanes
                    plsc.store_scatter(slab_ref, [rows * _LANES + col], v)

        P = zero_i
        A = zero_i
        k_rem = zero_i
        k_vec = zero_i
        for li, (ps, shift, nbins) in enumerate(
                ((31, 23, 256), (23, 15, 256), (15, 7, 256), (7, 0, 128))):

            @plsc.parallel_loop(0, nbins, unroll=8)
            def _(i):
                hist_ref[pl.ds(i * _LANES, _LANES)] = zero_i

            if li == 0:
                @plsc.parallel_loop(0, _B // _UNROLL, carry=zero_i)
                def k_vec(i, kk):
                    for j in range(_UNROLL):
                        v = slab_ref[pl.ds((i * _UNROLL + j) * _LANES,
                                           _LANES)]
                        m = v >= 0
                        bn = (v >> 23) & (nbins - 1)
                        plsc.addupdate_scatter(
                            hist_ref, [bn * _LANES + lanes], one_i, mask=m)
                        kk = kk + jnp.where(v < 0, one_i, zero_i)
                    return kk
                k_rem = k_vec
            else:
                @plsc.parallel_loop(0, _B // _UNROLL)
                def _(i, ps=ps, shift=shift, nbins=nbins, P=P):
                    for j in range(_UNROLL):
                        v = slab_ref[pl.ds((i * _UNROLL + j) * _LANES,
                                           _LANES)]
                        m = (v >> ps) == P
                        bn = (v >> shift) & (nbins - 1)
                        plsc.addupdate_scatter(
                            hist_ref, [bn * _LANES + lanes], one_i, mask=m)

            def scan_chunk(i, c, nbins=nbins, k_rem=k_rem):
                S, Al, bst, fnd = c
                for j in range(8):
                    bi_expr = nbins - 1 - (i * 8 + j)
                    h = hist_ref[pl.ds(bi_expr * _LANES, _LANES)]
                    Sn = S + h
                    crossing = (Sn >= k_rem) & (fnd == 0)
                    bst = jnp.where(crossing, bi_expr, bst)
                    Al = jnp.where(crossing, S, Al)
                    fnd = jnp.where(crossing, one_i, fnd)
                    S = Sn
                return (S, Al, bst, fnd)

            _, Al, bst, _ = lax.fori_loop(
                0, nbins // 8, scan_chunk,
                (zero_i, zero_i, zero_i, zero_i), unroll=False)

            k_rem = k_rem - Al
            A = A + Al
            P = (P << (ps - shift)) | bst

        T = P

        @plsc.parallel_loop(0, _B // _UNROLL, carry=zero_f)
        def s(i, acc):
            for j in range(_UNROLL):
                v = slab_ref[pl.ds((i * _UNROLL + j) * _LANES, _LANES)]
                gt = v > T
                acc = acc + jnp.where(gt, plsc.bitcast(v, jnp.float32),
                                      zero_f)
            return acc
        tie = plsc.bitcast(T, jnp.float32)
        extra = (k_vec - A).astype(jnp.float32) * tie
        res_ref[...] = jnp.where(k_vec > 0, s + extra, zero_f)
        pltpu.sync_copy(res_ref, out_hbm.at[pl.ds(c0, _LANES)])


_sc_stage = functools.partial(
    pl.kernel,
    out_type=jax.ShapeDtypeStruct((_CHALF,), jnp.float32),
    mesh=plsc.VectorSubcoreMesh(core_axis_name="c", subcore_axis_name="s"),
    compiler_params=pltpu.CompilerParams(needs_layout_passes=False),
    scratch_types=[
        pltpu.VMEM((_LANES, _HALF), jnp.int32),
        pltpu.VMEM((_B * _LANES,), jnp.int32),
        pltpu.VMEM((256 * _LANES,), jnp.int32),
        pltpu.VMEM((_LANES,), jnp.float32),
    ],
)(_sc_body)


def kernel(pred, target):
    bits0, mino0 = _tc_stage(pred, target, 0)
    topk0 = _sc_stage(bits0)
    bits1, mino1 = _tc_stage(pred, target, _CHALF // _BLK)
    topk1 = _sc_stage(bits1)
    return (mino0[0, 0] + mino1[0, 0] + jnp.sum(topk0) + jnp.sum(topk1)
            ) / jnp.float32(_B * _C)


# speedup vs baseline: 1.0761x; 1.0216x over previous
import functools

import jax
import jax.numpy as jnp
from jax import lax
from jax.experimental import pallas as pl
from jax.experimental.pallas import tpu as pltpu
from jax.experimental.pallas import tpu_sc as plsc

_B = 4096
_C = 1000
_CPAD = 1024
_BLK = 128
_NC = 2
_NS = 16
_NW = _NC * _NS
_LANES = 16
_CHALF = _CPAD // 2
_SLABS_PER_W = _CHALF // (_NW * _LANES)
_HALF = _B // 2
_UNROLL = 16


def _tc_body(pred_ref, targ_ref, bits_ref, mino_ref, *, j0):
    cls = ((pl.program_id(0) + j0) * _BLK
           + lax.broadcasted_iota(jnp.int32, (1, _BLK), 1))
    valid = cls < _C
    p = jnp.where(valid, pred_ref[...], 0.0)
    t = jnp.where(valid, targ_ref[...], 0.0)
    loss = jnp.maximum(p, 0.0) - p * t + jnp.log1p(jnp.exp(-jnp.abs(p)))

    pos = jnp.sum(t, axis=0, keepdims=True)
    pos_gt = (pos * 2.0 >= float(_B)).astype(jnp.float32)
    majority = t == pos_gt

    mino = jnp.sum(jnp.where(majority, 0.0, loss))
    bits = jnp.where(majority, lax.bitcast_convert_type(loss, jnp.int32),
                     jnp.int32(-1))
    bits_ref[...] = jnp.transpose(bits, (1, 0))

    @pl.when(pl.program_id(0) == 0)
    def _():
        mino_ref[...] = jnp.zeros((1, 1), jnp.float32)

    mino_ref[...] += jnp.reshape(mino, (1, 1))


def _tc_stage(predp, targp, j0):
    return pl.pallas_call(
        functools.partial(_tc_body, j0=j0),
        grid=(_CHALF // _BLK,),
        in_specs=[
            pl.BlockSpec((_B, _BLK), lambda j, j0=j0: (0, j + j0)),
            pl.BlockSpec((_B, _BLK), lambda j, j0=j0: (0, j + j0)),
        ],
        out_specs=[
            pl.BlockSpec((_BLK, _B), lambda j: (j, 0)),
            pl.BlockSpec((1, 1), lambda j: (0, 0)),
        ],
        out_shape=[
            jax.ShapeDtypeStruct((_CHALF, _B), jnp.int32),
            jax.ShapeDtypeStruct((1, 1), jnp.float32),
        ],
        compiler_params=pltpu.CompilerParams(
            dimension_semantics=("arbitrary",),
        ),
    )(predp, targp)


def _sc_body(bits_hbm, out_hbm, stage_ref, slab_ref, hist_ref, res_ref):
    wid = lax.axis_index("s") * _NC + lax.axis_index("c")
    lanes = jnp.arange(_LANES, dtype=jnp.int32)
    zero_i = jnp.zeros((_LANES,), jnp.int32)
    one_i = jnp.ones((_LANES,), jnp.int32)
    zero_f = jnp.zeros((_LANES,), jnp.float32)

    for slab in range(_SLABS_PER_W):
        c0 = (wid * _SLABS_PER_W + slab) * _LANES

        for h in range(2):
            pltpu.sync_copy(
                bits_hbm.at[pl.ds(c0, _LANES), pl.ds(h * _HALF, _HALF)],
                stage_ref)
            for c in range(_LANES):
                col = jnp.full((_LANES,), c, jnp.int32)

                @plsc.parallel_loop(0, _HALF // _LANES, unroll=8)
                def _(i, c=c, col=col, h=h):
                    v = stage_ref[c, pl.ds(i * _LANES, _LANES)]
                    rows = (h * _HALF + i * _LANES) + lanes
                    plsc.store_scatter(slab_ref, [rows * _LANES + col], v)

        P = zero_i
        A = zero_i
        k_rem = zero_i
        k_vec = zero_i
        for li, (ps, shift, nbins) in enumerate(
                ((31, 23, 256), (23, 15, 256), (15, 7, 256), (7, 0, 128))):

            @plsc.parallel_loop(0, nbins, unroll=8)
            def _(i):
                hist_ref[pl.ds(i * _LANES, _LANES)] = zero_i

            if li == 0:
                @plsc.parallel_loop(0, _B // _UNROLL, unroll=2, carry=zero_i)
                def k_vec(i, kk):
                    for j in range(_UNROLL):
                        v = slab_ref[pl.ds((i * _UNROLL + j) * _LANES,
                                           _LANES)]
                        m = v >= 0
                        bn = (v >> 23) & (nbins - 1)
                        plsc.addupdate_scatter(
                            hist_ref, [bn * _LANES + lanes], one_i, mask=m)
                        kk = kk + jnp.where(v < 0, one_i, zero_i)
                    return kk
                k_rem = k_vec
            else:
                @plsc.parallel_loop(0, _B // _UNROLL, unroll=2)
                def _(i, ps=ps, shift=shift, nbins=nbins, P=P):
                    for j in range(_UNROLL):
                        v = slab_ref[pl.ds((i * _UNROLL + j) * _LANES,
                                           _LANES)]
                        m = (v >> ps) == P
                        bn = (v >> shift) & (nbins - 1)
                        plsc.addupdate_scatter(
                            hist_ref, [bn * _LANES + lanes], one_i, mask=m)

            def scan_chunk(i, c, nbins=nbins, k_rem=k_rem):
                S, Al, bst, fnd = c
                for j in range(8):
                    bi_expr = nbins - 1 - (i * 8 + j)
                    h = hist_ref[pl.ds(bi_expr * _LANES, _LANES)]
                    Sn = S + h
                    crossing = (Sn >= k_rem) & (fnd == 0)
                    bst = jnp.where(crossing, bi_expr, bst)
                    Al = jnp.where(crossing, S, Al)
                    fnd = jnp.where(crossing, one_i, fnd)
                    S = Sn
                return (S, Al, bst, fnd)

            _, Al, bst, _ = lax.fori_loop(
                0, nbins // 8, scan_chunk,
                (zero_i, zero_i, zero_i, zero_i), unroll=False)

            k_rem = k_rem - Al
            A = A + Al
            P = (P << (ps - shift)) | bst

        T = P

        @plsc.parallel_loop(0, _B // _UNROLL, unroll=2, carry=zero_f)
        def s(i, acc):
            for j in range(_UNROLL):
                v = slab_ref[pl.ds((i * _UNROLL + j) * _LANES, _LANES)]
                gt = v > T
                acc = acc + jnp.where(gt, plsc.bitcast(v, jnp.float32),
                                      zero_f)
            return acc
        tie = plsc.bitcast(T, jnp.float32)
        extra = (k_vec - A).astype(jnp.float32) * tie
        res_ref[...] = jnp.where(k_vec > 0, s + extra, zero_f)
        pltpu.sync_copy(res_ref, out_hbm.at[pl.ds(c0, _LANES)])


_sc_stage = functools.partial(
    pl.kernel,
    out_type=jax.ShapeDtypeStruct((_CHALF,), jnp.float32),
    mesh=plsc.VectorSubcoreMesh(core_axis_name="c", subcore_axis_name="s"),
    compiler_params=pltpu.CompilerParams(needs_layout_passes=False),
    scratch_types=[
        pltpu.VMEM((_LANES, _HALF), jnp.int32),
        pltpu.VMEM((_B * _LANES,), jnp.int32),
        pltpu.VMEM((256 * _LANES,), jnp.int32),
        pltpu.VMEM((_LANES,), jnp.float32),
    ],
)(_sc_body)


def kernel(pred, target):
    bits0, mino0 = _tc_stage(pred, target, 0)
    topk0 = _sc_stage(bits0)
    bits1, mino1 = _tc_stage(pred, target, _CHALF // _BLK)
    topk1 = _sc_stage(bits1)
    return (mino0[0, 0] + mino1[0, 0] + jnp.sum(topk0) + jnp.sum(topk1)
            ) / jnp.float32(_B * _C)


# speedup vs baseline: 1.1064x; 1.0282x over previous
import functools

import jax
import jax.numpy as jnp
from jax import lax
from jax.experimental import pallas as pl
from jax.experimental.pallas import tpu as pltpu
from jax.experimental.pallas import tpu_sc as plsc

_B = 4096
_C = 1000
_CPAD = 1024
_BLK = 128
_NC = 2
_NS = 16
_NW = _NC * _NS
_LANES = 16
_CHALF = _CPAD // 2
_SLABS_PER_W = _CHALF // (_NW * _LANES)
_HALF = _B // 2
_UNROLL = 16


def _tc_body(pred_ref, targ_ref, bits_ref, mino_ref, *, j0):
    cls = ((pl.program_id(0) + j0) * _BLK
           + lax.broadcasted_iota(jnp.int32, (1, _BLK), 1))
    valid = cls < _C
    p = jnp.where(valid, pred_ref[...], 0.0)
    t = jnp.where(valid, targ_ref[...], 0.0)
    loss = jnp.maximum(p, 0.0) - p * t + jnp.log1p(jnp.exp(-jnp.abs(p)))

    pos = jnp.sum(t, axis=0, keepdims=True)
    pos_gt = (pos * 2.0 >= float(_B)).astype(jnp.float32)
    majority = t == pos_gt

    mino = jnp.sum(jnp.where(majority, 0.0, loss))
    bits = jnp.where(majority, lax.bitcast_convert_type(loss, jnp.int32),
                     jnp.int32(-1))
    bits_ref[...] = jnp.transpose(bits, (1, 0))

    @pl.when(pl.program_id(0) == 0)
    def _():
        mino_ref[...] = jnp.zeros((1, 1), jnp.float32)

    mino_ref[...] += jnp.reshape(mino, (1, 1))


def _tc_stage(predp, targp, j0):
    return pl.pallas_call(
        functools.partial(_tc_body, j0=j0),
        grid=(_CHALF // _BLK,),
        in_specs=[
            pl.BlockSpec((_B, _BLK), lambda j, j0=j0: (0, j + j0)),
            pl.BlockSpec((_B, _BLK), lambda j, j0=j0: (0, j + j0)),
        ],
        out_specs=[
            pl.BlockSpec((_BLK, _B), lambda j: (j, 0)),
            pl.BlockSpec((1, 1), lambda j: (0, 0)),
        ],
        out_shape=[
            jax.ShapeDtypeStruct((_CHALF, _B), jnp.int32),
            jax.ShapeDtypeStruct((1, 1), jnp.float32),
        ],
        compiler_params=pltpu.CompilerParams(
            dimension_semantics=("arbitrary",),
        ),
    )(predp, targp)


def _sc_body(bits_hbm, out_hbm, stage_ref, slab_ref, hist_ref, res_ref):
    wid = lax.axis_index("s") * _NC + lax.axis_index("c")
    lanes = jnp.arange(_LANES, dtype=jnp.int32)
    zero_i = jnp.zeros((_LANES,), jnp.int32)
    one_i = jnp.ones((_LANES,), jnp.int32)
    zero_f = jnp.zeros((_LANES,), jnp.float32)

    for slab in range(_SLABS_PER_W):
        c0 = (wid * _SLABS_PER_W + slab) * _LANES

        for h in range(2):
            pltpu.sync_copy(
                bits_hbm.at[pl.ds(c0, _LANES), pl.ds(h * _HALF, _HALF)],
                stage_ref)
            @plsc.parallel_loop(0, _HALF // _LANES, unroll=2)
            def _(i, h=h):
                for c in range(_LANES):
                    col = jnp.full((_LANES,), c, jnp.int32)
                    v = stage_ref[c, pl.ds(i * _LANES, _LANES)]
                    rows = (h * _HALF + i * _LANES) + lanes
                    plsc.store_scatter(slab_ref, [rows * _LANES + col], v)

        P = zero_i
        A = zero_i
        k_rem = zero_i
        k_vec = zero_i
        for li, (ps, shift, nbins) in enumerate(
                ((31, 23, 256), (23, 15, 256), (15, 7, 256), (7, 0, 128))):

            @plsc.parallel_loop(0, nbins, unroll=8)
            def _(i):
                hist_ref[pl.ds(i * _LANES, _LANES)] = zero_i

            if li == 0:
                @plsc.parallel_loop(0, _B // _UNROLL, unroll=4, carry=zero_i)
                def k_vec(i, kk):
                    for j in range(_UNROLL):
                        v = slab_ref[pl.ds((i * _UNROLL + j) * _LANES,
                                           _LANES)]
                        m = v >= 0
                        bn = (v >> 23) & (nbins - 1)
                        plsc.addupdate_scatter(
                            hist_ref, [bn * _LANES + lanes], one_i, mask=m)
                        kk = kk + jnp.where(v < 0, one_i, zero_i)
                    return kk
                k_rem = k_vec
            else:
                @plsc.parallel_loop(0, _B // _UNROLL, unroll=4)
                def _(i, ps=ps, shift=shift, nbins=nbins, P=P):
                    for j in range(_UNROLL):
                        v = slab_ref[pl.ds((i * _UNROLL + j) * _LANES,
                                           _LANES)]
                        m = (v >> ps) == P
                        bn = (v >> shift) & (nbins - 1)
                        plsc.addupdate_scatter(
                            hist_ref, [bn * _LANES + lanes], one_i, mask=m)

            def scan_chunk(i, c, nbins=nbins, k_rem=k_rem):
                S, Al, bst, fnd = c
                for j in range(8):
                    bi_expr = nbins - 1 - (i * 8 + j)
                    h = hist_ref[pl.ds(bi_expr * _LANES, _LANES)]
                    Sn = S + h
                    crossing = (Sn >= k_rem) & (fnd == 0)
                    bst = jnp.where(crossing, bi_expr, bst)
                    Al = jnp.where(crossing, S, Al)
                    fnd = jnp.where(crossing, one_i, fnd)
                    S = Sn
                return (S, Al, bst, fnd)

            _, Al, bst, _ = lax.fori_loop(
                0, nbins // 8, scan_chunk,
                (zero_i, zero_i, zero_i, zero_i), unroll=False)

            k_rem = k_rem - Al
            A = A + Al
            P = (P << (ps - shift)) | bst

        T = P

        @plsc.parallel_loop(0, _B // _UNROLL, unroll=4, carry=zero_f)
        def s(i, acc):
            for j in range(_UNROLL):
                v = slab_ref[pl.ds((i * _UNROLL + j) * _LANES, _LANES)]
                gt = v > T
                acc = acc + jnp.where(gt, plsc.bitcast(v, jnp.float32),
                                      zero_f)
            return acc
        tie = plsc.bitcast(T, jnp.float32)
        extra = (k_vec - A).astype(jnp.float32) * tie
        res_ref[...] = jnp.where(k_vec > 0, s + extra, zero_f)
        pltpu.sync_copy(res_ref, out_hbm.at[pl.ds(c0, _LANES)])


_sc_stage = functools.partial(
    pl.kernel,
    out_type=jax.ShapeDtypeStruct((_CHALF,), jnp.float32),
    mesh=plsc.VectorSubcoreMesh(core_axis_name="c", subcore_axis_name="s"),
    compiler_params=pltpu.CompilerParams(needs_layout_passes=False),
    scratch_types=[
        pltpu.VMEM((_LANES, _HALF), jnp.int32),
        pltpu.VMEM((_B * _LANES,), jnp.int32),
        pltpu.VMEM((256 * _LANES,), jnp.int32),
        pltpu.VMEM((_LANES,), jnp.float32),
    ],
)(_sc_body)


def kernel(pred, target):
    bits0, mino0 = _tc_stage(pred, target, 0)
    topk0 = _sc_stage(bits0)
    bits1, mino1 = _tc_stage(pred, target, _CHALF // _BLK)
    topk1 = _sc_stage(bits1)
    return (mino0[0, 0] + mino1[0, 0] + jnp.sum(topk0) + jnp.sum(topk1)
            ) / jnp.float32(_B * _C)
